# baseline (speedup 1.0000x reference)
import functools

import jax
import jax.numpy as jnp
from jax import lax
from jax.experimental import pallas as pl
from jax.experimental.pallas import tpu as pltpu
from jax.experimental.pallas import tpu_sc as plsc

N = 10000
E = 320000
IN_CH = 128
HID_CH = 256
OUT_CH = 128
BN_EPS = 1e-5

NC = 2
NS = 16
EPS_SC = E // NS
REAL = 125
CHUNK = 128
NCHUNK = EPS_SC // REAL
NACC = 10016
RPT = NACC // NS
DH = IN_CH // NC
CW = 16


def _make_segsum(with_counts):
    mesh = plsc.VectorSubcoreMesh(core_axis_name="c", subcore_axis_name="s")

    out_type = [jax.ShapeDtypeStruct((NACC, IN_CH), jnp.float32)]
    scratch = [
        pltpu.VMEM((NCHUNK, CHUNK), jnp.int32),
        pltpu.VMEM((NCHUNK, CHUNK), jnp.int32),
        pltpu.VMEM((3, CHUNK, DH), jnp.float32),
        pltpu.VMEM_SHARED((NACC, DH), jnp.float32),
        pltpu.SemaphoreType.DMA,
        pltpu.SemaphoreType.DMA,
    ]
    if with_counts:
        out_type.append(jax.ShapeDtypeStruct((NACC, CW), jnp.float32))
        scratch += [
            pltpu.VMEM((CHUNK, CW), jnp.float32),
            pltpu.VMEM_SHARED((NACC, CW), jnp.float32),
        ]

    @functools.partial(
        pl.kernel,
        out_type=tuple(out_type),
        mesh=mesh,
        compiler_params=pltpu.CompilerParams(use_tc_tiling_on_sc=False),
        scratch_types=scratch,
    )
    def seg(x_hbm, src_hbm, dst_hbm, zeros_hbm, zeros8_hbm,
            ones8_hbm, out_hbm, *rest):
        if with_counts:
            (cnt_hbm, src_v, dst_v, rows3, acc, gsem, ssem,
             ones_v, acc2) = rest
        else:
            src_v, dst_v, rows3, acc, gsem, ssem = rest
        c = lax.axis_index("c")
        s = lax.axis_index("s")
        xc = x_hbm
        pltpu.sync_copy(zeros_hbm, acc.at[pl.ds(s * RPT, RPT)])
        pltpu.sync_copy(src_hbm.at[s], src_v)
        pltpu.sync_copy(dst_hbm.at[s], dst_v)

        def dbl(i, carry):
            for q in range(CHUNK // 16):
                sl = pl.ds(q * 16, 16)
                src_v[i, sl] = src_v[i, sl] * 2 + c
            return carry

        lax.fori_loop(0, NCHUNK, dbl, 0)
        if with_counts:
            pltpu.sync_copy(ones8_hbm, ones_v)
            pltpu.sync_copy(zeros8_hbm, acc2.at[pl.ds(s * RPT, RPT)])
        plsc.subcore_barrier()

        def body(i, carry):
            @pl.when(i < NCHUNK)
            def _():
                pltpu.async_copy(xc.at[src_v.at[i]],
                                 rows3.at[lax.rem(i, 3)], gsem)

            @pl.when(jnp.logical_and(i >= 1, i <= NCHUNK))
            def _():
                pltpu.make_async_copy(zeros_hbm.at[pl.ds(0, CHUNK)],
                                      rows3.at[0], gsem).wait()
                j = i - 1
                pltpu.async_copy(rows3.at[lax.rem(j, 3)],
                                 acc.at[dst_v.at[j]], ssem, add=True)
                if with_counts:
                    pltpu.sync_copy(ones_v, acc2.at[dst_v.at[j]],
                                    add=True)

            @pl.when(i >= 2)
            def _():
                pltpu.make_async_copy(zeros_hbm.at[pl.ds(0, CHUNK)],
                                      acc.at[pl.ds(0, CHUNK)], ssem).wait()

            return carry

        lax.fori_loop(0, NCHUNK + 2, body, 0)
        plsc.subcore_barrier()
        pltpu.sync_copy(acc.at[pl.ds(s * RPT, RPT)],
                        out_hbm.at[pl.ds(s * RPT, RPT), pl.ds(c * DH, DH)])

        if with_counts:
            @pl.when(c == 0)
            def _():
                pltpu.sync_copy(acc2.at[pl.ds(s * RPT, RPT)],
                                cnt_hbm.at[pl.ds(s * RPT, RPT)])


    return seg


_seg_cnt = _make_segsum(True)
_seg_plain = _make_segsum(False)


def _phase2(sums, cnts, x, W1_l, b1, W1_r, gamma, beta, W2_l, b2, W2_r):
    def body(sum_ref, cnt_ref, x_ref, w1l_ref, b1_ref, w1r_ref, g_ref,
             be_ref, w2l_ref, b2_ref, w2r_ref, p_ref, r_ref):
        cnt = cnt_ref[0:N, 0:1]
        rinv = 1.0 / jnp.maximum(cnt, 1.0)
        agg = sum_ref[0:N, :] * rinv
        h = (jnp.dot(agg, w1l_ref[...], preferred_element_type=jnp.float32)
             + b1_ref[...]
             + jnp.dot(x_ref[...], w1r_ref[...],
                       preferred_element_type=jnp.float32))
        mu = jnp.mean(h, axis=0, keepdims=True)
        var = jnp.mean((h - mu) ** 2, axis=0, keepdims=True)
        hn = (h - mu) / jnp.sqrt(var + BN_EPS) * g_ref[...] + be_ref[...]
        hr = jnp.maximum(hn, 0.0)
        p_ref[...] = jnp.dot(hr, w2l_ref[...],
                             preferred_element_type=jnp.float32)
        r_ref[...] = jnp.dot(hr, w2r_ref[...],
                             preferred_element_type=jnp.float32) + b2_ref[...]

    return pl.pallas_call(
        body,
        out_shape=(
            jax.ShapeDtypeStruct((N, OUT_CH), jnp.float32),
            jax.ShapeDtypeStruct((N, OUT_CH), jnp.float32),
        ),
    )(sums, cnts, x, W1_l, b1.reshape(1, HID_CH), W1_r,
      gamma.reshape(1, HID_CH), beta.reshape(1, HID_CH),
      W2_l, b2.reshape(1, OUT_CH), W2_r)


def _phase4(sums2, cnts, r):
    def body(sum_ref, cnt_ref, r_ref, out_ref):
        rinv = 1.0 / jnp.maximum(cnt_ref[0:N, 0:1], 1.0)
        out_ref[...] = sum_ref[0:N, :] * rinv + r_ref[...]

    return pl.pallas_call(
        body,
        out_shape=jax.ShapeDtypeStruct((N, OUT_CH), jnp.float32),
    )(sums2, cnts, r)


def kernel(x, edge_index, W1_l, b1, W1_r, gamma, beta, W2_l, b2, W2_r):
    pad3 = ((0, 0), (0, 0), (0, CHUNK - REAL))
    src = jnp.pad(edge_index[0].reshape(NS, NCHUNK, REAL), pad3)
    dst = jnp.pad(edge_index[1].reshape(NS, NCHUNK, REAL), pad3,
                  constant_values=N)
    zeros = jnp.zeros((RPT, DH), jnp.float32)
    zeros8 = jnp.zeros((RPT, CW), jnp.float32)
    ones8 = jnp.ones((CHUNK, CW), jnp.float32)

    xh = x.reshape(2 * N, DH)
    sums1, cnts = _seg_cnt(xh, src, dst, zeros, zeros8, ones8)
    p, r = _phase2(sums1, cnts, x, W1_l, b1, W1_r, gamma, beta,
                   W2_l, b2, W2_r)
    ph = p.reshape(2 * N, DH)
    (sums2,) = _seg_plain(ph, src, dst, zeros, zeros8, ones8)
    return _phase4(sums2, cnts, r)


# speedup vs baseline: 2.7548x; 2.7548x over previous
import functools

import jax
import jax.numpy as jnp
from jax import lax
from jax.experimental import pallas as pl
from jax.experimental.pallas import tpu as pltpu
from jax.experimental.pallas import tpu_sc as plsc

N = 10000
E = 320000
IN_CH = 128
HID_CH = 256
OUT_CH = 128
BN_EPS = 1e-5

NC = 2
NS = 16
EPS_SC = E // NS
CHUNK = 125
NCHUNK = EPS_SC // CHUNK
RPT = N // NS
DH = IN_CH // NC
CW = 16


def _make_segsum(with_counts):
    mesh = plsc.VectorSubcoreMesh(core_axis_name="c", subcore_axis_name="s")

    out_type = [jax.ShapeDtypeStruct((N, IN_CH), jnp.float32)]
    scratch = [
        pltpu.VMEM((NCHUNK, CHUNK), jnp.int32),
        pltpu.VMEM((NCHUNK, CHUNK), jnp.int32),
        pltpu.VMEM((3, CHUNK, DH), jnp.float32),
        pltpu.VMEM_SHARED((N, DH), jnp.float32),
        pltpu.SemaphoreType.DMA,
        pltpu.SemaphoreType.DMA,
    ]
    if with_counts:
        out_type.append(jax.ShapeDtypeStruct((N, CW), jnp.float32))
        scratch += [
            pltpu.VMEM((CHUNK, CW), jnp.float32),
            pltpu.VMEM_SHARED((N, CW), jnp.float32),
        ]

    @functools.partial(
        pl.kernel,
        out_type=tuple(out_type),
        mesh=mesh,
        compiler_params=pltpu.CompilerParams(use_tc_tiling_on_sc=False),
        scratch_types=scratch,
    )
    def seg(x_hbm, src0_hbm, src1_hbm, dst_hbm, zeros_hbm, zeros8_hbm,
            ones8_hbm, out_hbm, *rest):
        if with_counts:
            (cnt_hbm, src_v, dst_v, rows3, acc, gsem, ssem,
             ones_v, acc2) = rest
        else:
            src_v, dst_v, rows3, acc, gsem, ssem = rest
        c = lax.axis_index("c")
        s = lax.axis_index("s")
        xc = x_hbm
        pltpu.sync_copy(zeros_hbm.at[pl.ds(s * RPT, RPT)],
                        acc.at[pl.ds(s * RPT, RPT)])
        @pl.when(c == 0)
        def _():
            pltpu.sync_copy(src0_hbm.at[s], src_v)

        @pl.when(c == 1)
        def _():
            pltpu.sync_copy(src1_hbm.at[s], src_v)

        pltpu.sync_copy(dst_hbm.at[s], dst_v)
        if with_counts:
            pltpu.sync_copy(ones8_hbm, ones_v)
            pltpu.sync_copy(zeros8_hbm.at[pl.ds(s * RPT, RPT)],
                            acc2.at[pl.ds(s * RPT, RPT)])
        plsc.subcore_barrier()

        def body(i, carry):
            @pl.when(i < NCHUNK)
            def _():
                pltpu.async_copy(xc.at[src_v.at[i]],
                                 rows3.at[lax.rem(i, 3)], gsem)

            @pl.when(jnp.logical_and(i >= 1, i <= NCHUNK))
            def _():
                pltpu.make_async_copy(zeros_hbm.at[pl.ds(0, CHUNK)],
                                      rows3.at[0], gsem).wait()
                j = i - 1
                pltpu.async_copy(rows3.at[lax.rem(j, 3)],
                                 acc.at[dst_v.at[j]], ssem, add=True)
                if with_counts:
                    pltpu.sync_copy(ones_v, acc2.at[dst_v.at[j]],
                                    add=True)

            @pl.when(i >= 2)
            def _():
                pltpu.make_async_copy(zeros_hbm.at[pl.ds(0, CHUNK)],
                                      acc.at[pl.ds(0, CHUNK)], ssem).wait()

            return carry

        lax.fori_loop(0, NCHUNK + 2, body, 0)
        plsc.subcore_barrier()
        pltpu.sync_copy(acc.at[pl.ds(s * RPT, RPT)],
                        out_hbm.at[pl.ds(s * RPT, RPT), pl.ds(c * DH, DH)])
        if with_counts:
            @pl.when(c == 0)
            def _():
                pltpu.sync_copy(acc2.at[pl.ds(s * RPT, RPT)],
                                cnt_hbm.at[pl.ds(s * RPT, RPT)])

    return seg


_seg_cnt = _make_segsum(True)
_seg_plain = _make_segsum(False)


def _phase2(sums, cnts, x, W1_l, b1, W1_r, gamma, beta, W2_l, b2, W2_r):
    def body(sum_ref, cnt_ref, x_ref, w1l_ref, b1_ref, w1r_ref, g_ref,
             be_ref, w2l_ref, b2_ref, w2r_ref, p_ref, r_ref):
        cnt = cnt_ref[:, 0:1]
        rinv = 1.0 / jnp.maximum(cnt, 1.0)
        agg = sum_ref[...] * rinv
        h = (jnp.dot(agg, w1l_ref[...], preferred_element_type=jnp.float32)
             + b1_ref[...]
             + jnp.dot(x_ref[...], w1r_ref[...],
                       preferred_element_type=jnp.float32))
        mu = jnp.mean(h, axis=0, keepdims=True)
        var = jnp.mean((h - mu) ** 2, axis=0, keepdims=True)
        hn = (h - mu) / jnp.sqrt(var + BN_EPS) * g_ref[...] + be_ref[...]
        hr = jnp.maximum(hn, 0.0)
        p_ref[...] = jnp.dot(hr, w2l_ref[...],
                             preferred_element_type=jnp.float32)
        r_ref[...] = jnp.dot(hr, w2r_ref[...],
                             preferred_element_type=jnp.float32) + b2_ref[...]

    return pl.pallas_call(
        body,
        out_shape=(
            jax.ShapeDtypeStruct((N, OUT_CH), jnp.float32),
            jax.ShapeDtypeStruct((N, OUT_CH), jnp.float32),
        ),
    )(sums, cnts, x, W1_l, b1.reshape(1, HID_CH), W1_r,
      gamma.reshape(1, HID_CH), beta.reshape(1, HID_CH),
      W2_l, b2.reshape(1, OUT_CH), W2_r)


def _phase4(sums2, cnts, r):
    def body(sum_ref, cnt_ref, r_ref, out_ref):
        rinv = 1.0 / jnp.maximum(cnt_ref[:, 0:1], 1.0)
        out_ref[...] = sum_ref[...] * rinv + r_ref[...]

    return pl.pallas_call(
        body,
        out_shape=jax.ShapeDtypeStruct((N, OUT_CH), jnp.float32),
    )(sums2, cnts, r)


def kernel(x, edge_index, W1_l, b1, W1_r, gamma, beta, W2_l, b2, W2_r):
    src0 = edge_index[0].reshape(NS, NCHUNK, CHUNK) * 2
    src1 = src0 + 1
    dst = edge_index[1].reshape(NS, NCHUNK, CHUNK)
    zeros = jnp.zeros((2 * N, DH), jnp.float32)
    zeros8 = jnp.zeros((N, CW), jnp.float32)
    ones8 = jnp.ones((CHUNK, CW), jnp.float32)

    xh = x.reshape(2 * N, DH)
    sums1, cnts = _seg_cnt(xh, src0, src1, dst, zeros, zeros8, ones8)
    p, r = _phase2(sums1, cnts, x, W1_l, b1, W1_r, gamma, beta,
                   W2_l, b2, W2_r)
    ph = p.reshape(2 * N, DH)
    (sums2,) = _seg_plain(ph, src0, src1, dst, zeros, zeros8, ones8)
    return _phase4(sums2, cnts, r)


# speedup vs baseline: 2.8096x; 1.0199x over previous
import functools

import jax
import jax.numpy as jnp
from jax import lax
from jax.experimental import pallas as pl
from jax.experimental.pallas import tpu as pltpu
from jax.experimental.pallas import tpu_sc as plsc

N = 10000
E = 320000
IN_CH = 128
HID_CH = 256
OUT_CH = 128
BN_EPS = 1e-5

NC = 2
NS = 16
EPS_SC = E // NS
CHUNK = 125
NCHUNK = EPS_SC // CHUNK
RPT = N // NS
DH = IN_CH // NC
CW = 8


def _make_segsum(with_counts):
    mesh = plsc.VectorSubcoreMesh(core_axis_name="c", subcore_axis_name="s")

    out_type = [jax.ShapeDtypeStruct((N, IN_CH), jnp.float32)]
    scratch = [
        pltpu.VMEM((NCHUNK, CHUNK), jnp.int32),
        pltpu.VMEM((NCHUNK, CHUNK), jnp.int32),
        pltpu.VMEM((3, CHUNK, DH), jnp.float32),
        pltpu.VMEM_SHARED((N, DH), jnp.float32),
        pltpu.SemaphoreType.DMA,
        pltpu.SemaphoreType.DMA,
    ]
    if with_counts:
        out_type.append(jax.ShapeDtypeStruct((N, CW), jnp.float32))
        scratch += [
            pltpu.VMEM((CHUNK, CW), jnp.float32),
            pltpu.VMEM_SHARED((N, CW), jnp.float32),
        ]

    @functools.partial(
        pl.kernel,
        out_type=tuple(out_type),
        mesh=mesh,
        compiler_params=pltpu.CompilerParams(use_tc_tiling_on_sc=False),
        scratch_types=scratch,
    )
    def seg(x_hbm, src0_hbm, src1_hbm, dst_hbm, zeros_hbm, zeros8_hbm,
            ones8_hbm, out_hbm, *rest):
        if with_counts:
            (cnt_hbm, src_v, dst_v, rows3, acc, gsem, ssem,
             ones_v, acc2) = rest
        else:
            src_v, dst_v, rows3, acc, gsem, ssem = rest
        c = lax.axis_index("c")
        s = lax.axis_index("s")
        xc = x_hbm
        pltpu.sync_copy(zeros_hbm.at[pl.ds(s * RPT, RPT)],
                        acc.at[pl.ds(s * RPT, RPT)])
        @pl.when(c == 0)
        def _():
            pltpu.sync_copy(src0_hbm.at[s], src_v)

        @pl.when(c == 1)
        def _():
            pltpu.sync_copy(src1_hbm.at[s], src_v)

        pltpu.sync_copy(dst_hbm.at[s], dst_v)
        if with_counts:
            pltpu.sync_copy(ones8_hbm, ones_v)
            pltpu.sync_copy(zeros8_hbm.at[pl.ds(s * RPT, RPT)],
                            acc2.at[pl.ds(s * RPT, RPT)])
        plsc.subcore_barrier()

        def body(i, carry):
            @pl.when(i < NCHUNK)
            def _():
                pltpu.async_copy(xc.at[src_v.at[i]],
                                 rows3.at[lax.rem(i, 3)], gsem)

            @pl.when(jnp.logical_and(i >= 1, i <= NCHUNK))
            def _():
                pltpu.make_async_copy(zeros_hbm.at[pl.ds(0, CHUNK)],
                                      rows3.at[0], gsem).wait()
                j = i - 1
                pltpu.async_copy(rows3.at[lax.rem(j, 3)],
                                 acc.at[dst_v.at[j]], ssem, add=True)
                if with_counts:
                    pltpu.sync_copy(ones_v, acc2.at[dst_v.at[j]],
                                    add=True)

            @pl.when(i >= 2)
            def _():
                pltpu.make_async_copy(zeros_hbm.at[pl.ds(0, CHUNK)],
                                      acc.at[pl.ds(0, CHUNK)], ssem).wait()

            return carry

        lax.fori_loop(0, NCHUNK + 2, body, 0)
        plsc.subcore_barrier()
        pltpu.sync_copy(acc.at[pl.ds(s * RPT, RPT)],
                        out_hbm.at[pl.ds(s * RPT, RPT), pl.ds(c * DH, DH)])
        if with_counts:
            @pl.when(c == 0)
            def _():
                pltpu.sync_copy(acc2.at[pl.ds(s * RPT, RPT)],
                                cnt_hbm.at[pl.ds(s * RPT, RPT)])

    return seg


_seg_cnt = _make_segsum(True)
_seg_plain = _make_segsum(False)


def _phase2(sums, cnts, x, W1_l, b1, W1_r, gamma, beta, W2_l, b2, W2_r):
    def body(sum_ref, cnt_ref, x_ref, w1l_ref, b1_ref, w1r_ref, g_ref,
             be_ref, w2l_ref, b2_ref, w2r_ref, p_ref, r_ref):
        cnt = cnt_ref[:, 0:1]
        rinv = 1.0 / jnp.maximum(cnt, 1.0)
        agg = sum_ref[...] * rinv
        h = (jnp.dot(agg, w1l_ref[...], preferred_element_type=jnp.float32)
             + b1_ref[...]
             + jnp.dot(x_ref[...], w1r_ref[...],
                       preferred_element_type=jnp.float32))
        mu = jnp.mean(h, axis=0, keepdims=True)
        var = jnp.mean((h - mu) ** 2, axis=0, keepdims=True)
        hn = (h - mu) / jnp.sqrt(var + BN_EPS) * g_ref[...] + be_ref[...]
        hr = jnp.maximum(hn, 0.0)
        p_ref[...] = jnp.dot(hr, w2l_ref[...],
                             preferred_element_type=jnp.float32)
        r_ref[...] = jnp.dot(hr, w2r_ref[...],
                             preferred_element_type=jnp.float32) + b2_ref[...]

    return pl.pallas_call(
        body,
        out_shape=(
            jax.ShapeDtypeStruct((N, OUT_CH), jnp.float32),
            jax.ShapeDtypeStruct((N, OUT_CH), jnp.float32),
        ),
    )(sums, cnts, x, W1_l, b1.reshape(1, HID_CH), W1_r,
      gamma.reshape(1, HID_CH), beta.reshape(1, HID_CH),
      W2_l, b2.reshape(1, OUT_CH), W2_r)


def _phase4(sums2, cnts, r):
    def body(sum_ref, cnt_ref, r_ref, out_ref):
        rinv = 1.0 / jnp.maximum(cnt_ref[:, 0:1], 1.0)
        out_ref[...] = sum_ref[...] * rinv + r_ref[...]

    return pl.pallas_call(
        body,
        out_shape=jax.ShapeDtypeStruct((N, OUT_CH), jnp.float32),
    )(sums2, cnts, r)


def kernel(x, edge_index, W1_l, b1, W1_r, gamma, beta, W2_l, b2, W2_r):
    src0 = edge_index[0].reshape(NS, NCHUNK, CHUNK) * 2
    src1 = src0 + 1
    dst = edge_index[1].reshape(NS, NCHUNK, CHUNK)
    zeros = jnp.zeros((2 * N, DH), jnp.float32)
    zeros8 = jnp.zeros((N, CW), jnp.float32)
    ones8 = jnp.ones((CHUNK, CW), jnp.float32)

    xh = x.reshape(2 * N, DH)
    sums1, cnts = _seg_cnt(xh, src0, src1, dst, zeros, zeros8, ones8)
    p, r = _phase2(sums1, cnts, x, W1_l, b1, W1_r, gamma, beta,
                   W2_l, b2, W2_r)
    ph = p.reshape(2 * N, DH)
    (sums2,) = _seg_plain(ph, src0, src1, dst, zeros, zeros8, ones8)
    return _phase4(sums2, cnts, r)


# speedup vs baseline: 2.8117x; 1.0008x over previous
import functools

import jax
import jax.numpy as jnp
from jax import lax
from jax.experimental import pallas as pl
from jax.experimental.pallas import tpu as pltpu
from jax.experimental.pallas import tpu_sc as plsc

N = 10000
E = 320000
IN_CH = 128
HID_CH = 256
OUT_CH = 128
BN_EPS = 1e-5

NC = 2
NS = 16
EPS_SC = E // NS
CHUNK = 125
NCHUNK = EPS_SC // CHUNK
RPT = N // NS
DH = IN_CH // NC
CW = 8


def _make_segsum(with_counts):
    mesh = plsc.VectorSubcoreMesh(core_axis_name="c", subcore_axis_name="s")

    out_type = [jax.ShapeDtypeStruct((N, IN_CH), jnp.float32)]
    scratch = [
        pltpu.VMEM((NCHUNK, CHUNK), jnp.int32),
        pltpu.VMEM((NCHUNK, CHUNK), jnp.int32),
        pltpu.VMEM((3, CHUNK, DH), jnp.float32),
        pltpu.VMEM_SHARED((N, DH), jnp.float32),
        pltpu.SemaphoreType.DMA,
        pltpu.SemaphoreType.DMA,
    ]
    if with_counts:
        out_type.append(jax.ShapeDtypeStruct((N, CW), jnp.float32))
        scratch += [
            pltpu.VMEM((CHUNK, CW), jnp.float32),
            pltpu.VMEM_SHARED((N, CW), jnp.float32),
        ]

    @functools.partial(
        pl.kernel,
        out_type=tuple(out_type),
        mesh=mesh,
        compiler_params=pltpu.CompilerParams(use_tc_tiling_on_sc=False),
        scratch_types=scratch,
    )
    def seg(x_hbm, src0_hbm, src1_hbm, dst_hbm, zeros_hbm, zeros8_hbm,
            ones8_hbm, out_hbm, *rest):
        if with_counts:
            (cnt_hbm, src_v, dst_v, rows3, acc, gsem, ssem,
             ones_v, acc2) = rest
        else:
            src_v, dst_v, rows3, acc, gsem, ssem = rest
        c = lax.axis_index("c")
        s = lax.axis_index("s")
        xc = x_hbm
        pltpu.sync_copy(zeros_hbm.at[pl.ds(s * RPT, RPT)],
                        acc.at[pl.ds(s * RPT, RPT)])
        @pl.when(c == 0)
        def _():
            pltpu.sync_copy(src0_hbm.at[s], src_v)

        @pl.when(c == 1)
        def _():
            pltpu.sync_copy(src1_hbm.at[s], src_v)

        pltpu.sync_copy(dst_hbm.at[s], dst_v)
        if with_counts:
            pltpu.sync_copy(ones8_hbm, ones_v)
            pltpu.sync_copy(zeros8_hbm.at[pl.ds(s * RPT, RPT)],
                            acc2.at[pl.ds(s * RPT, RPT)])
        plsc.subcore_barrier()

        def body(i, carry):
            @pl.when(i < NCHUNK)
            def _():
                pltpu.async_copy(xc.at[src_v.at[i]],
                                 rows3.at[lax.rem(i, 3)], gsem)

            @pl.when(jnp.logical_and(i >= 1, i <= NCHUNK))
            def _():
                pltpu.make_async_copy(zeros_hbm.at[pl.ds(0, CHUNK)],
                                      rows3.at[0], gsem).wait()
                j = i - 1
                pltpu.async_copy(rows3.at[lax.rem(j, 3)],
                                 acc.at[dst_v.at[j]], ssem, add=True)
                if with_counts:
                    pltpu.sync_copy(ones_v, acc2.at[dst_v.at[j]],
                                    add=True)

            @pl.when(i >= 2)
            def _():
                pltpu.make_async_copy(zeros_hbm.at[pl.ds(0, CHUNK)],
                                      acc.at[pl.ds(0, CHUNK)], ssem).wait()

            return carry

        lax.fori_loop(0, NCHUNK + 2, body, 0)
        plsc.subcore_barrier()
        pltpu.sync_copy(acc.at[pl.ds(s * RPT, RPT)],
                        out_hbm.at[pl.ds(s * RPT, RPT), pl.ds(c * DH, DH)])
        if with_counts:
            @pl.when(c == 0)
            def _():
                pltpu.sync_copy(acc2.at[pl.ds(s * RPT, RPT)],
                                cnt_hbm.at[pl.ds(s * RPT, RPT)])

    return seg


_seg_cnt = _make_segsum(True)
_seg_plain = _make_segsum(False)


def _phase2(sums, cnts, x, W1_l, b1, W1_r, gamma, beta, W2_l, b2, W2_r):
    def body(sum_ref, cnt_ref, x_ref, w1l_ref, b1_ref, w1r_ref, g_ref,
             be_ref, w2l_ref, b2_ref, w2r_ref, p_ref, r_ref):
        cnt = cnt_ref[:, 0:1]
        rinv = 1.0 / jnp.maximum(cnt, 1.0)
        agg = sum_ref[...] * rinv
        h = (jnp.dot(agg.astype(jnp.bfloat16),
                     w1l_ref[...].astype(jnp.bfloat16),
                     preferred_element_type=jnp.float32)
             + b1_ref[...]
             + jnp.dot(x_ref[...].astype(jnp.bfloat16),
                       w1r_ref[...].astype(jnp.bfloat16),
                       preferred_element_type=jnp.float32))
        mu = jnp.mean(h, axis=0, keepdims=True)
        var = jnp.mean((h - mu) ** 2, axis=0, keepdims=True)
        hn = (h - mu) / jnp.sqrt(var + BN_EPS) * g_ref[...] + be_ref[...]
        hr = jnp.maximum(hn, 0.0)
        hb = hr.astype(jnp.bfloat16)
        p_ref[...] = jnp.dot(hb, w2l_ref[...].astype(jnp.bfloat16),
                             preferred_element_type=jnp.float32)
        r_ref[...] = jnp.dot(hb, w2r_ref[...].astype(jnp.bfloat16),
                             preferred_element_type=jnp.float32) + b2_ref[...]

    return pl.pallas_call(
        body,
        out_shape=(
            jax.ShapeDtypeStruct((N, OUT_CH), jnp.float32),
            jax.ShapeDtypeStruct((N, OUT_CH), jnp.float32),
        ),
    )(sums, cnts, x, W1_l, b1.reshape(1, HID_CH), W1_r,
      gamma.reshape(1, HID_CH), beta.reshape(1, HID_CH),
      W2_l, b2.reshape(1, OUT_CH), W2_r)


def _phase4(sums2, cnts, r):
    def body(sum_ref, cnt_ref, r_ref, out_ref):
        rinv = 1.0 / jnp.maximum(cnt_ref[:, 0:1], 1.0)
        out_ref[...] = sum_ref[...] * rinv + r_ref[...]

    return pl.pallas_call(
        body,
        out_shape=jax.ShapeDtypeStruct((N, OUT_CH), jnp.float32),
    )(sums2, cnts, r)


def kernel(x, edge_index, W1_l, b1, W1_r, gamma, beta, W2_l, b2, W2_r):
    src0 = edge_index[0].reshape(NS, NCHUNK, CHUNK) * 2
    src1 = src0 + 1
    dst = edge_index[1].reshape(NS, NCHUNK, CHUNK)
    zeros = jnp.zeros((2 * N, DH), jnp.float32)
    zeros8 = jnp.zeros((N, CW), jnp.float32)
    ones8 = jnp.ones((CHUNK, CW), jnp.float32)

    xh = x.reshape(2 * N, DH)
    sums1, cnts = _seg_cnt(xh, src0, src1, dst, zeros, zeros8, ones8)
    p, r = _phase2(sums1, cnts, x, W1_l, b1, W1_r, gamma, beta,
                   W2_l, b2, W2_r)
    ph = p.reshape(2 * N, DH)
    (sums2,) = _seg_plain(ph, src0, src1, dst, zeros, zeros8, ones8)
    return _phase4(sums2, cnts, r)


# speedup vs baseline: 2.8842x; 1.0258x over previous
import functools

import jax
import jax.numpy as jnp
from jax import lax
from jax.experimental import pallas as pl
from jax.experimental.pallas import tpu as pltpu
from jax.experimental.pallas import tpu_sc as plsc

N = 10000
E = 320000
IN_CH = 128
HID_CH = 256
OUT_CH = 128
BN_EPS = 1e-5

NC = 2
NS = 16
EPS_SC = E // NS
CHUNK = 125
NCHUNK = EPS_SC // CHUNK
RPT = N // NS
DH = IN_CH // NC
CW = 8


def _make_segsum(with_counts):
    mesh = plsc.VectorSubcoreMesh(core_axis_name="c", subcore_axis_name="s")

    out_type = [jax.ShapeDtypeStruct((N, IN_CH), jnp.bfloat16)]
    scratch = [
        pltpu.VMEM((NCHUNK, CHUNK), jnp.int32),
        pltpu.VMEM((NCHUNK, CHUNK), jnp.int32),
        pltpu.VMEM((3, CHUNK, DH), jnp.bfloat16),
        pltpu.VMEM_SHARED((N, DH), jnp.bfloat16),
        pltpu.SemaphoreType.DMA,
        pltpu.SemaphoreType.DMA,
    ]
    if with_counts:
        out_type.append(jax.ShapeDtypeStruct((N, CW), jnp.float32))
        scratch += [
            pltpu.VMEM((CHUNK, CW), jnp.float32),
            pltpu.VMEM_SHARED((N, CW), jnp.float32),
        ]

    @functools.partial(
        pl.kernel,
        out_type=tuple(out_type),
        mesh=mesh,
        compiler_params=pltpu.CompilerParams(use_tc_tiling_on_sc=False),
        scratch_types=scratch,
    )
    def seg(x_hbm, src0_hbm, src1_hbm, dst_hbm, zeros_hbm, zeros8_hbm,
            ones8_hbm, out_hbm, *rest):
        if with_counts:
            (cnt_hbm, src_v, dst_v, rows3, acc, gsem, ssem,
             ones_v, acc2) = rest
        else:
            src_v, dst_v, rows3, acc, gsem, ssem = rest
        c = lax.axis_index("c")
        s = lax.axis_index("s")
        xc = x_hbm
        pltpu.sync_copy(zeros_hbm.at[pl.ds(s * RPT, RPT)],
                        acc.at[pl.ds(s * RPT, RPT)])
        @pl.when(c == 0)
        def _():
            pltpu.sync_copy(src0_hbm.at[s], src_v)

        @pl.when(c == 1)
        def _():
            pltpu.sync_copy(src1_hbm.at[s], src_v)

        pltpu.sync_copy(dst_hbm.at[s], dst_v)
        if with_counts:
            pltpu.sync_copy(ones8_hbm, ones_v)
            pltpu.sync_copy(zeros8_hbm.at[pl.ds(s * RPT, RPT)],
                            acc2.at[pl.ds(s * RPT, RPT)])
        plsc.subcore_barrier()

        def body(i, carry):
            @pl.when(i < NCHUNK)
            def _():
                pltpu.async_copy(xc.at[src_v.at[i]],
                                 rows3.at[lax.rem(i, 3)], gsem)

            @pl.when(jnp.logical_and(i >= 1, i <= NCHUNK))
            def _():
                pltpu.make_async_copy(zeros_hbm.at[pl.ds(0, CHUNK)],
                                      rows3.at[0], gsem).wait()
                j = i - 1
                pltpu.async_copy(rows3.at[lax.rem(j, 3)],
                                 acc.at[dst_v.at[j]], ssem, add=True)
                if with_counts:
                    pltpu.sync_copy(ones_v, acc2.at[dst_v.at[j]],
                                    add=True)

            @pl.when(i >= 2)
            def _():
                pltpu.make_async_copy(zeros_hbm.at[pl.ds(0, CHUNK)],
                                      acc.at[pl.ds(0, CHUNK)], ssem).wait()

            return carry

        lax.fori_loop(0, NCHUNK + 2, body, 0)
        plsc.subcore_barrier()
        pltpu.sync_copy(acc.at[pl.ds(s * RPT, RPT)],
                        out_hbm.at[pl.ds(s * RPT, RPT), pl.ds(c * DH, DH)])
        if with_counts:
            @pl.when(c == 0)
            def _():
                pltpu.sync_copy(acc2.at[pl.ds(s * RPT, RPT)],
                                cnt_hbm.at[pl.ds(s * RPT, RPT)])

    return seg


_seg_cnt = _make_segsum(True)
_seg_plain = _make_segsum(False)


def _phase2(sums, cnts, x, W1_l, b1, W1_r, gamma, beta, W2_l, b2, W2_r):
    def body(sum_ref, cnt_ref, x_ref, w1l_ref, b1_ref, w1r_ref, g_ref,
             be_ref, w2l_ref, b2_ref, w2r_ref, p_ref, r_ref):
        cnt = cnt_ref[:, 0:1]
        rinv = 1.0 / jnp.maximum(cnt, 1.0)
        agg = sum_ref[...].astype(jnp.float32) * rinv
        h = (jnp.dot(agg, w1l_ref[...], preferred_element_type=jnp.float32)
             + b1_ref[...]
             + jnp.dot(x_ref[...], w1r_ref[...],
                       preferred_element_type=jnp.float32))
        mu = jnp.mean(h, axis=0, keepdims=True)
        var = jnp.mean((h - mu) ** 2, axis=0, keepdims=True)
        hn = (h - mu) / jnp.sqrt(var + BN_EPS) * g_ref[...] + be_ref[...]
        hr = jnp.maximum(hn, 0.0)
        p_ref[...] = jnp.dot(hr, w2l_ref[...],
                             preferred_element_type=jnp.float32
                             ).astype(jnp.bfloat16)
        r_ref[...] = jnp.dot(hr, w2r_ref[...],
                             preferred_element_type=jnp.float32) + b2_ref[...]

    return pl.pallas_call(
        body,
        out_shape=(
            jax.ShapeDtypeStruct((N, OUT_CH), jnp.bfloat16),
            jax.ShapeDtypeStruct((N, OUT_CH), jnp.float32),
        ),
    )(sums, cnts, x, W1_l, b1.reshape(1, HID_CH), W1_r,
      gamma.reshape(1, HID_CH), beta.reshape(1, HID_CH),
      W2_l, b2.reshape(1, OUT_CH), W2_r)


def _phase4(sums2, cnts, r):
    def body(sum_ref, cnt_ref, r_ref, out_ref):
        rinv = 1.0 / jnp.maximum(cnt_ref[:, 0:1], 1.0)
        out_ref[...] = sum_ref[...].astype(jnp.float32) * rinv + r_ref[...]

    return pl.pallas_call(
        body,
        out_shape=jax.ShapeDtypeStruct((N, OUT_CH), jnp.float32),
    )(sums2, cnts, r)


def kernel(x, edge_index, W1_l, b1, W1_r, gamma, beta, W2_l, b2, W2_r):
    src0 = edge_index[0].reshape(NS, NCHUNK, CHUNK) * 2
    src1 = src0 + 1
    dst = edge_index[1].reshape(NS, NCHUNK, CHUNK)
    zeros = jnp.zeros((2 * N, DH), jnp.bfloat16)
    zeros8 = jnp.zeros((N, CW), jnp.float32)
    ones8 = jnp.ones((CHUNK, CW), jnp.float32)

    xh = x.astype(jnp.bfloat16).reshape(2 * N, DH)
    sums1, cnts = _seg_cnt(xh, src0, src1, dst, zeros, zeros8, ones8)
    p, r = _phase2(sums1, cnts, x, W1_l, b1, W1_r, gamma, beta,
                   W2_l, b2, W2_r)
    ph = p.reshape(2 * N, DH)
    (sums2,) = _seg_plain(ph, src0, src1, dst, zeros, zeros8, ones8)
    return _phase4(sums2, cnts, r)


# speedup vs baseline: 3.3476x; 1.1607x over previous
import functools

import jax
import jax.numpy as jnp
from jax import lax
from jax.experimental import pallas as pl
from jax.experimental.pallas import tpu as pltpu
from jax.experimental.pallas import tpu_sc as plsc

N = 10000
E = 320000
IN_CH = 128
HID_CH = 256
OUT_CH = 128
BN_EPS = 1e-5

NC = 2
NS = 16
EPS_SC = E // NS
CHUNK = 125
NCHUNK = EPS_SC // CHUNK
RPT = N // NS
DH = IN_CH // NC
CW = 8


def _make_segsum(with_counts):
    mesh = plsc.VectorSubcoreMesh(core_axis_name="c", subcore_axis_name="s")

    out_type = [jax.ShapeDtypeStruct((N, IN_CH), jnp.bfloat16)]
    scratch = [
        pltpu.VMEM((NCHUNK, CHUNK), jnp.int32),
        pltpu.VMEM((NCHUNK, CHUNK), jnp.int32),
        pltpu.VMEM((6, CHUNK, DH), jnp.bfloat16),
        pltpu.VMEM_SHARED((N, DH), jnp.bfloat16),
        pltpu.SemaphoreType.DMA,
        pltpu.SemaphoreType.DMA,
    ]
    if with_counts:
        out_type.append(jax.ShapeDtypeStruct((N, CW), jnp.float32))
        scratch += [
            pltpu.VMEM((CHUNK, CW), jnp.float32),
            pltpu.VMEM_SHARED((N, CW), jnp.float32),
        ]

    @functools.partial(
        pl.kernel,
        out_type=tuple(out_type),
        mesh=mesh,
        compiler_params=pltpu.CompilerParams(use_tc_tiling_on_sc=False),
        scratch_types=scratch,
    )
    def seg(x_hbm, src0_hbm, src1_hbm, dst_hbm, zeros_hbm, zeros8_hbm,
            ones8_hbm, out_hbm, *rest):
        if with_counts:
            (cnt_hbm, src_v, dst_v, rows3, acc, gsem, ssem,
             ones_v, acc2) = rest
        else:
            src_v, dst_v, rows3, acc, gsem, ssem = rest
        c = lax.axis_index("c")
        s = lax.axis_index("s")
        xc = x_hbm
        pltpu.sync_copy(zeros_hbm.at[pl.ds(s * RPT, RPT)],
                        acc.at[pl.ds(s * RPT, RPT)])
        @pl.when(c == 0)
        def _():
            pltpu.sync_copy(src0_hbm.at[s], src_v)

        @pl.when(c == 1)
        def _():
            pltpu.sync_copy(src1_hbm.at[s], src_v)

        pltpu.sync_copy(dst_hbm.at[s], dst_v)
        if with_counts:
            pltpu.sync_copy(ones8_hbm, ones_v)
            pltpu.sync_copy(zeros8_hbm.at[pl.ds(s * RPT, RPT)],
                            acc2.at[pl.ds(s * RPT, RPT)])
        plsc.subcore_barrier()

        def body(i, carry):
            @pl.when(i < NCHUNK)
            def _():
                pltpu.async_copy(xc.at[src_v.at[i]],
                                 rows3.at[lax.rem(i, 6)], gsem)

            @pl.when(jnp.logical_and(i >= 2, i <= NCHUNK + 1))
            def _():
                pltpu.make_async_copy(zeros_hbm.at[pl.ds(0, CHUNK)],
                                      rows3.at[0], gsem).wait()
                j = i - 2
                pltpu.async_copy(rows3.at[lax.rem(j, 6)],
                                 acc.at[dst_v.at[j]], ssem, add=True)
                if with_counts:
                    pltpu.sync_copy(ones_v, acc2.at[dst_v.at[j]],
                                    add=True)

            @pl.when(i >= 5)
            def _():
                pltpu.make_async_copy(zeros_hbm.at[pl.ds(0, CHUNK)],
                                      acc.at[pl.ds(0, CHUNK)], ssem).wait()

            return carry

        lax.fori_loop(0, NCHUNK + 5, body, 0)
        plsc.subcore_barrier()
        pltpu.sync_copy(acc.at[pl.ds(s * RPT, RPT)],
                        out_hbm.at[pl.ds(s * RPT, RPT), pl.ds(c * DH, DH)])
        if with_counts:
            @pl.when(c == 0)
            def _():
                pltpu.sync_copy(acc2.at[pl.ds(s * RPT, RPT)],
                                cnt_hbm.at[pl.ds(s * RPT, RPT)])

    return seg


_seg_cnt = _make_segsum(True)
_seg_plain = _make_segsum(False)


def _phase2(sums, cnts, x, W1_l, b1, W1_r, gamma, beta, W2_l, b2, W2_r):
    def body(sum_ref, cnt_ref, x_ref, w1l_ref, b1_ref, w1r_ref, g_ref,
             be_ref, w2l_ref, b2_ref, w2r_ref, p_ref, r_ref):
        cnt = cnt_ref[:, 0:1]
        rinv = 1.0 / jnp.maximum(cnt, 1.0)
        agg = sum_ref[...].astype(jnp.float32) * rinv
        h = (jnp.dot(agg, w1l_ref[...], preferred_element_type=jnp.float32)
             + b1_ref[...]
             + jnp.dot(x_ref[...], w1r_ref[...],
                       preferred_element_type=jnp.float32))
        mu = jnp.mean(h, axis=0, keepdims=True)
        var = jnp.mean((h - mu) ** 2, axis=0, keepdims=True)
        hn = (h - mu) / jnp.sqrt(var + BN_EPS) * g_ref[...] + be_ref[...]
        hr = jnp.maximum(hn, 0.0)
        p_ref[...] = jnp.dot(hr, w2l_ref[...],
                             preferred_element_type=jnp.float32
                             ).astype(jnp.bfloat16)
        r_ref[...] = jnp.dot(hr, w2r_ref[...],
                             preferred_element_type=jnp.float32) + b2_ref[...]

    return pl.pallas_call(
        body,
        out_shape=(
            jax.ShapeDtypeStruct((N, OUT_CH), jnp.bfloat16),
            jax.ShapeDtypeStruct((N, OUT_CH), jnp.float32),
        ),
    )(sums, cnts, x, W1_l, b1.reshape(1, HID_CH), W1_r,
      gamma.reshape(1, HID_CH), beta.reshape(1, HID_CH),
      W2_l, b2.reshape(1, OUT_CH), W2_r)


def _phase4(sums2, cnts, r):
    def body(sum_ref, cnt_ref, r_ref, out_ref):
        rinv = 1.0 / jnp.maximum(cnt_ref[:, 0:1], 1.0)
        out_ref[...] = sum_ref[...].astype(jnp.float32) * rinv + r_ref[...]

    return pl.pallas_call(
        body,
        out_shape=jax.ShapeDtypeStruct((N, OUT_CH), jnp.float32),
    )(sums2, cnts, r)


def kernel(x, edge_index, W1_l, b1, W1_r, gamma, beta, W2_l, b2, W2_r):
    src0 = edge_index[0].reshape(NS, NCHUNK, CHUNK) * 2
    src1 = src0 + 1
    dst = edge_index[1].reshape(NS, NCHUNK, CHUNK)
    zeros = jnp.zeros((2 * N, DH), jnp.bfloat16)
    zeros8 = jnp.zeros((N, CW), jnp.float32)
    ones8 = jnp.ones((CHUNK, CW), jnp.float32)

    xh = x.astype(jnp.bfloat16).reshape(2 * N, DH)
    sums1, cnts = _seg_cnt(xh, src0, src1, dst, zeros, zeros8, ones8)
    p, r = _phase2(sums1, cnts, x, W1_l, b1, W1_r, gamma, beta,
                   W2_l, b2, W2_r)
    ph = p.reshape(2 * N, DH)
    (sums2,) = _seg_plain(ph, src0, src1, dst, zeros, zeros8, ones8)
    return _phase4(sums2, cnts, r)


# speedup vs baseline: 3.6689x; 1.0960x over previous
import functools

import jax
import jax.numpy as jnp
from jax import lax
from jax.experimental import pallas as pl
from jax.experimental.pallas import tpu as pltpu
from jax.experimental.pallas import tpu_sc as plsc

N = 10000
E = 320000
IN_CH = 128
HID_CH = 256
OUT_CH = 128
BN_EPS = 1e-5

NC = 2
NS = 16
EPS_SC = E // NS
CHUNK = 125
NCHUNK = EPS_SC // CHUNK
RPT = N // NS
DH = IN_CH // NC
CW = 8


def _make_segsum(with_counts):
    mesh = plsc.VectorSubcoreMesh(core_axis_name="c", subcore_axis_name="s")

    out_type = [jax.ShapeDtypeStruct((N, IN_CH), jnp.bfloat16)]
    scratch = [
        pltpu.VMEM((NCHUNK, CHUNK), jnp.int32),
        pltpu.VMEM((NCHUNK, CHUNK), jnp.int32),
        pltpu.VMEM((10, CHUNK, DH), jnp.bfloat16),
        pltpu.VMEM_SHARED((N, DH), jnp.bfloat16),
        pltpu.SemaphoreType.DMA,
        pltpu.SemaphoreType.DMA,
    ]
    if with_counts:
        out_type.append(jax.ShapeDtypeStruct((N, CW), jnp.float32))
        scratch += [
            pltpu.VMEM((CHUNK, CW), jnp.float32),
            pltpu.VMEM_SHARED((N, CW), jnp.float32),
        ]

    @functools.partial(
        pl.kernel,
        out_type=tuple(out_type),
        mesh=mesh,
        compiler_params=pltpu.CompilerParams(use_tc_tiling_on_sc=False),
        scratch_types=scratch,
    )
    def seg(x_hbm, src0_hbm, src1_hbm, dst_hbm, zeros_hbm, zeros8_hbm,
            ones8_hbm, out_hbm, *rest):
        if with_counts:
            (cnt_hbm, src_v, dst_v, rows3, acc, gsem, ssem,
             ones_v, acc2) = rest
        else:
            src_v, dst_v, rows3, acc, gsem, ssem = rest
        c = lax.axis_index("c")
        s = lax.axis_index("s")
        xc = x_hbm
        pltpu.sync_copy(zeros_hbm.at[pl.ds(s * RPT, RPT)],
                        acc.at[pl.ds(s * RPT, RPT)])
        @pl.when(c == 0)
        def _():
            pltpu.sync_copy(src0_hbm.at[s], src_v)

        @pl.when(c == 1)
        def _():
            pltpu.sync_copy(src1_hbm.at[s], src_v)

        pltpu.sync_copy(dst_hbm.at[s], dst_v)
        if with_counts:
            pltpu.sync_copy(ones8_hbm, ones_v)
            pltpu.sync_copy(zeros8_hbm.at[pl.ds(s * RPT, RPT)],
                            acc2.at[pl.ds(s * RPT, RPT)])
        plsc.subcore_barrier()

        def body(i, carry):
            @pl.when(i < NCHUNK)
            def _():
                pltpu.async_copy(xc.at[src_v.at[i]],
                                 rows3.at[lax.rem(i, 10)], gsem)

            @pl.when(jnp.logical_and(i >= 4, i <= NCHUNK + 3))
            def _():
                pltpu.make_async_copy(zeros_hbm.at[pl.ds(0, CHUNK)],
                                      rows3.at[0], gsem).wait()
                j = i - 4
                pltpu.async_copy(rows3.at[lax.rem(j, 10)],
                                 acc.at[dst_v.at[j]], ssem, add=True)
                if with_counts:
                    pltpu.sync_copy(ones_v, acc2.at[dst_v.at[j]],
                                    add=True)

            @pl.when(i >= 9)
            def _():
                pltpu.make_async_copy(zeros_hbm.at[pl.ds(0, CHUNK)],
                                      acc.at[pl.ds(0, CHUNK)], ssem).wait()

            return carry

        lax.fori_loop(0, NCHUNK + 9, body, 0)
        plsc.subcore_barrier()
        pltpu.sync_copy(acc.at[pl.ds(s * RPT, RPT)],
                        out_hbm.at[pl.ds(s * RPT, RPT), pl.ds(c * DH, DH)])
        if with_counts:
            @pl.when(c == 0)
            def _():
                pltpu.sync_copy(acc2.at[pl.ds(s * RPT, RPT)],
                                cnt_hbm.at[pl.ds(s * RPT, RPT)])

    return seg


_seg_cnt = _make_segsum(True)
_seg_plain = _make_segsum(False)


def _phase2(sums, cnts, x, W1_l, b1, W1_r, gamma, beta, W2_l, b2, W2_r):
    def body(sum_ref, cnt_ref, x_ref, w1l_ref, b1_ref, w1r_ref, g_ref,
             be_ref, w2l_ref, b2_ref, w2r_ref, p_ref, r_ref):
        cnt = cnt_ref[:, 0:1]
        rinv = 1.0 / jnp.maximum(cnt, 1.0)
        agg = sum_ref[...].astype(jnp.float32) * rinv
        h = (jnp.dot(agg, w1l_ref[...], preferred_element_type=jnp.float32)
             + b1_ref[...]
             + jnp.dot(x_ref[...], w1r_ref[...],
                       preferred_element_type=jnp.float32))
        mu = jnp.mean(h, axis=0, keepdims=True)
        var = jnp.mean((h - mu) ** 2, axis=0, keepdims=True)
        hn = (h - mu) / jnp.sqrt(var + BN_EPS) * g_ref[...] + be_ref[...]
        hr = jnp.maximum(hn, 0.0)
        p_ref[...] = jnp.dot(hr, w2l_ref[...],
                             preferred_element_type=jnp.float32
                             ).astype(jnp.bfloat16)
        r_ref[...] = jnp.dot(hr, w2r_ref[...],
                             preferred_element_type=jnp.float32) + b2_ref[...]

    return pl.pallas_call(
        body,
        out_shape=(
            jax.ShapeDtypeStruct((N, OUT_CH), jnp.bfloat16),
            jax.ShapeDtypeStruct((N, OUT_CH), jnp.float32),
        ),
    )(sums, cnts, x, W1_l, b1.reshape(1, HID_CH), W1_r,
      gamma.reshape(1, HID_CH), beta.reshape(1, HID_CH),
      W2_l, b2.reshape(1, OUT_CH), W2_r)


def _phase4(sums2, cnts, r):
    def body(sum_ref, cnt_ref, r_ref, out_ref):
        rinv = 1.0 / jnp.maximum(cnt_ref[:, 0:1], 1.0)
        out_ref[...] = sum_ref[...].astype(jnp.float32) * rinv + r_ref[...]

    return pl.pallas_call(
        body,
        out_shape=jax.ShapeDtypeStruct((N, OUT_CH), jnp.float32),
    )(sums2, cnts, r)


def kernel(x, edge_index, W1_l, b1, W1_r, gamma, beta, W2_l, b2, W2_r):
    src0 = edge_index[0].reshape(NS, NCHUNK, CHUNK) * 2
    src1 = src0 + 1
    dst = edge_index[1].reshape(NS, NCHUNK, CHUNK)
    zeros = jnp.zeros((2 * N, DH), jnp.bfloat16)
    zeros8 = jnp.zeros((N, CW), jnp.float32)
    ones8 = jnp.ones((CHUNK, CW), jnp.float32)

    xh = x.astype(jnp.bfloat16).reshape(2 * N, DH)
    sums1, cnts = _seg_cnt(xh, src0, src1, dst, zeros, zeros8, ones8)
    p, r = _phase2(sums1, cnts, x, W1_l, b1, W1_r, gamma, beta,
                   W2_l, b2, W2_r)
    ph = p.reshape(2 * N, DH)
    (sums2,) = _seg_plain(ph, src0, src1, dst, zeros, zeros8, ones8)
    return _phase4(sums2, cnts, r)
